# baseline (speedup 1.0000x reference)
import functools
import math

import jax
import jax.numpy as jnp
from jax import lax
from jax.experimental import pallas as pl
from jax.experimental.pallas import tpu as pltpu
from jax.experimental.pallas import tpu_sc as plsc

D_MODEL = 64
SCALE = math.sqrt(D_MODEL)


@functools.partial(jax.jit, static_argnames=("N", "W", "D", "H"))
def _emb_lookup(xT, table3, *, N, W, D, H):
    info = plsc.get_sparse_core_info()
    NC, NS, L = info.num_cores, info.num_subcores, info.num_lanes
    NW = NC * NS
    NB = N // 128
    n_blocks = W * NB
    assert n_blocks % NW == 0
    blocks_per_w = n_blocks // NW
    assert D % L == 0

    mesh = plsc.VectorSubcoreMesh(core_axis_name="c", subcore_axis_name="s")

    @functools.partial(
        pl.kernel,
        mesh=mesh,
        compiler_params=pltpu.CompilerParams(needs_layout_passes=False),
        out_type=jax.ShapeDtypeStruct((W, D, N), jnp.float32),
        scratch_types=[
            pltpu.VMEM((128,), jnp.int32),
            pltpu.VMEM((128, D), jnp.float32),
            pltpu.VMEM((D, 128), jnp.float32),
            pltpu.SemaphoreType.DMA,
        ],
    )
    def k(xT_hbm, tp_hbm, out_hbm, idxb_v, rows_v, blkT_v, sem):
        wid = lax.axis_index("s") * NC + lax.axis_index("c")
        bid0 = wid * blocks_per_w
        lanes = lax.iota(jnp.int32, L)

        def block(kk, carry0):
            bid = bid0 + kk
            w = lax.shift_right_logical(bid, 5)
            nb = lax.bitwise_and(bid, jnp.int32(NB - 1))
            col0 = nb * 128
            pltpu.sync_copy(xT_hbm.at[w, pl.ds(col0, 128)], idxb_v)

            def issue(r, carry):
                vec = idxb_v[pl.ds(r * L, L)]
                hi = jnp.where(vec >= H, jnp.int32(1), jnp.int32(0))
                lo = vec - hi * H
                for t in range(L):
                    pltpu.make_async_copy(
                        tp_hbm.at[hi[t], lo[t]], rows_v.at[r * L + t], sem
                    ).start()
                return carry

            lax.fori_loop(0, 128 // L, issue, 0)
            pltpu.make_async_copy(
                tp_hbm.at[0, pl.ds(0, 128)], rows_v, sem
            ).wait()

            def tpose(n, carry):
                ncol = jnp.full((L,), n, jnp.int32)
                for g in range(D // L):
                    v = rows_v[n, pl.ds(g * L, L)] * SCALE
                    plsc.store_scatter(blkT_v, [lanes + g * L, ncol], v)
                return carry

            lax.fori_loop(0, 128, tpose, 0)

            pltpu.sync_copy(blkT_v, out_hbm.at[w, :, pl.ds(col0, 128)])
            return carry0

        lax.fori_loop(0, blocks_per_w, block, 0)

    return k(xT, table3)


def kernel(x, word_emb_weight):
    N, W = x.shape
    V, D = word_emb_weight.shape
    xT = x.T
    table3 = word_emb_weight.reshape(2, V // 2, D)
    out = _emb_lookup(xT, table3, N=N, W=W, D=D, H=V // 2)
    return jnp.transpose(out, (2, 0, 1))


# speedup vs baseline: 1.0691x; 1.0691x over previous
import functools
import math

import jax
import jax.numpy as jnp
from jax import lax
from jax.experimental import pallas as pl
from jax.experimental.pallas import tpu as pltpu
from jax.experimental.pallas import tpu_sc as plsc

D_MODEL = 64
SCALE = math.sqrt(D_MODEL)


@functools.partial(jax.jit, static_argnames=("N", "W", "D", "H"))
def _emb_lookup(idxT_flat, table3, *, N, W, D, H):
    info = plsc.get_sparse_core_info()
    NC, NS, L = info.num_cores, info.num_subcores, info.num_lanes
    NW = NC * NS
    B = N * W
    assert B % NW == 0
    b_per_w = B // NW
    NB = N // 128
    CB = 2
    C = CB * 128
    n_chunks = b_per_w // C
    assert D % L == 0

    mesh = plsc.VectorSubcoreMesh(core_axis_name="c", subcore_axis_name="s")

    @functools.partial(
        pl.kernel,
        mesh=mesh,
        compiler_params=pltpu.CompilerParams(needs_layout_passes=False),
        out_type=jax.ShapeDtypeStruct((W, D, N), jnp.float32),
        scratch_types=[
            pltpu.VMEM((b_per_w,), jnp.int32),
            pltpu.VMEM((C, D), jnp.float32),
            pltpu.VMEM((D, C), jnp.float32),
            pltpu.SemaphoreType.DMA,
        ],
    )
    def k(idx_hbm, tp_hbm, out_hbm, idx_v, rows_v, blkT_v, sem):
        wid = lax.axis_index("s") * NC + lax.axis_index("c")
        base = wid * b_per_w
        bid0 = wid * (b_per_w // 128)
        lanes = lax.iota(jnp.int32, L)
        pltpu.sync_copy(idx_hbm.at[pl.ds(base, b_per_w)], idx_v)

        def chunk(j, carry0):
            def issue(r, carry):
                vec = idx_v[pl.ds(j * C + r * L, L)]
                hi = jnp.where(vec >= H, jnp.int32(1), jnp.int32(0))
                lo = vec - hi * H
                for t in range(L):
                    pltpu.make_async_copy(
                        tp_hbm.at[hi[t], lo[t]], rows_v.at[r * L + t], sem
                    ).start()
                return carry

            lax.fori_loop(0, C // L, issue, 0)
            pltpu.make_async_copy(
                tp_hbm.at[0, pl.ds(0, C)], rows_v, sem
            ).wait()

            def tpose(n, carry):
                ncol = jnp.full((L,), n, jnp.int32)
                for g in range(D // L):
                    v = rows_v[n, pl.ds(g * L, L)] * SCALE
                    plsc.store_scatter(blkT_v, [lanes + g * L, ncol], v)
                return carry

            lax.fori_loop(0, C, tpose, 0)

            bid = bid0 + j * CB
            w = lax.shift_right_logical(bid, 5)
            nb = lax.bitwise_and(bid, jnp.int32(NB - 1))
            pltpu.sync_copy(
                blkT_v, out_hbm.at[w, :, pl.ds(nb * 128, C)]
            )
            return carry0

        lax.fori_loop(0, n_chunks, chunk, 0)

    return k(idxT_flat, table3)


def kernel(x, word_emb_weight):
    N, W = x.shape
    V, D = word_emb_weight.shape
    idxT_flat = x.T.reshape(N * W)
    table3 = word_emb_weight.reshape(2, V // 2, D)
    out = _emb_lookup(idxT_flat, table3, N=N, W=W, D=D, H=V // 2)
    return jnp.transpose(out, (2, 0, 1))
